# baseline (speedup 1.0000x reference)
import functools

import jax
import jax.numpy as jnp
from jax.experimental import pallas as pl

TOPK = 8
E = 64
BM = 1024
BC = 128
NEG = -3.0e38


def _select_chunk(lt, b, rows):
    scores = jax.nn.sigmoid(lt)
    biased = scores + b
    idx_parts = []
    w_parts = []
    cur = biased
    for _ in range(TOPK):
        m = jnp.max(cur, axis=0, keepdims=True)
        is_max = cur == m
        idx_k = jnp.min(jnp.where(is_max, rows, float(E)), axis=0, keepdims=True)
        sel = rows == idx_k
        w_k = jnp.sum(jnp.where(sel, scores, 0.0), axis=0, keepdims=True)
        idx_parts.append(idx_k)
        w_parts.append(w_k)
        cur = jnp.where(sel, NEG, cur)

    w = jnp.concatenate(w_parts, axis=0)
    idx = jnp.concatenate(idx_parts, axis=0)
    w = w / (jnp.sum(w, axis=0, keepdims=True) + 1e-20)
    return w, idx.astype(jnp.int32)


def _router_kernel(x_ref, wt_ref, b_ref, w_out_ref, i_out_ref):
    logits = jnp.dot(x_ref[...], wt_ref[...], preferred_element_type=jnp.float32)
    b = b_ref[...]
    rows = jax.lax.broadcasted_iota(jnp.int32, (E, BC), 0).astype(jnp.float32)
    for c in range(BM // BC):
        lt = logits[c * BC:(c + 1) * BC, :].T
        w, idx = _select_chunk(lt, b, rows)
        w_out_ref[:, c * BC:(c + 1) * BC] = w
        i_out_ref[:, c * BC:(c + 1) * BC] = idx


@functools.partial(jax.jit, static_argnames=())
def kernel(x, weight, bias):
    t = x.shape[0]
    wt = weight.T
    bt = bias.reshape(E, 1)
    grid = (t // BM,)
    w, idx = pl.pallas_call(
        _router_kernel,
        grid=grid,
        in_specs=[
            pl.BlockSpec((BM, x.shape[1]), lambda i: (i, 0)),
            pl.BlockSpec((x.shape[1], E), lambda i: (0, 0)),
            pl.BlockSpec((E, 1), lambda i: (0, 0)),
        ],
        out_specs=[
            pl.BlockSpec((TOPK, BM), lambda i: (0, i)),
            pl.BlockSpec((TOPK, BM), lambda i: (0, i)),
        ],
        out_shape=[
            jax.ShapeDtypeStruct((TOPK, t), jnp.float32),
            jax.ShapeDtypeStruct((TOPK, t), jnp.int32),
        ],
    )(x, wt, bt)
    return w.T, idx.T
